# baseline (speedup 1.0000x reference)
import functools

import jax
import jax.numpy as jnp
from jax import lax
from jax.experimental import pallas as pl
from jax.experimental.pallas import tpu as pltpu
from jax.experimental.pallas import tpu_sc as plsc

_NUM_CORES = 2
_NUM_SUBCORES = 16
_NUM_WORKERS = _NUM_CORES * _NUM_SUBCORES
_LANES = 16
_BLK = 128


def _pick_chunk(n_edges: int) -> int:
    for c in range(2560, _BLK - 1, -_BLK):
        if n_edges % c == 0:
            return c
    return _BLK


def _sc_body(pos4_hbm, ei_hbm, ox_hbm, oy_hbm, oz_hbm,
             idx_v, rows_v, ox_v, oy_v, oz_v,
             sem_idx, sem_row, sem_out,
             *, chunk: int, n_chunks: int):
    wid = lax.axis_index("s") * _NUM_CORES + lax.axis_index("c")
    n_mine = (n_chunks - wid + _NUM_WORKERS - 1) // _NUM_WORKERS
    grp_per_blk = _BLK // _LANES

    lane_iota = lax.iota(jnp.int32, _LANES)
    comp = [jnp.full((_LANES,), j, jnp.int32) for j in range(3)]
    half = jnp.float32(0.5)
    threehalf = jnp.float32(1.5)
    magic = jnp.int32(0x5F3759DF)

    def idx_half(j):
        return idx_v.at[pl.ds((j % 2) * chunk * 2, chunk * 2)]

    def rows_half(j):
        return rows_v.at[pl.ds((j % 2) * chunk * 2, chunk * 2), :]

    def ei_src(j):
        off = (wid + j * _NUM_WORKERS) * chunk
        return ei_hbm.at[pl.ds(off * 2, chunk * 2)]

    def out_parts(j):
        o = pl.ds((j % 2) * chunk, chunk)
        off = (wid + j * _NUM_WORKERS) * chunk
        h = pl.ds(off, chunk)
        return ((ox_v.at[o], ox_hbm.at[h]), (oy_v.at[o], oy_hbm.at[h]),
                (oz_v.at[o], oz_hbm.at[h]))

    def idx_copy(j):
        return pltpu.make_async_copy(ei_src(j), idx_half(j),
                                     sem_idx.at[j % 2])

    def row_copy(j):
        return pltpu.make_async_copy(pos4_hbm.at[idx_half(j)], rows_half(j),
                                     sem_row.at[j % 2])

    idx_copy(0).start()
    idx_copy(1).start()
    idx_copy(0).wait()
    row_copy(0).start()

    def chunk_body(j, _):
        sel = (j % 2) * chunk * 2

        @pl.when(j + 1 < n_mine)
        def _fire_next_gather():
            idx_copy(j + 1).wait()
            row_copy(j + 1).start()

        row_copy(j).wait()

        @pl.when(j + 2 < n_mine)
        def _fire_next_idx():
            idx_copy(j + 2).start()

        @pl.when(j >= 2)
        def _drain_out():
            for src, dst in out_parts(j - 2):
                pltpu.make_async_copy(src, dst, sem_out.at[j % 2]).wait()

        osel = (j % 2) * chunk

        def blk_body(b, _):
            r0 = sel + b * (2 * _BLK)
            e0b = osel + b * _BLK
            for t in range(grp_per_blk):
                eid_r = lane_iota + (r0 + t * _LANES)
                eid_c = eid_r + _BLK
                rx = plsc.load_gather(rows_v, [eid_r, comp[0]])
                ry = plsc.load_gather(rows_v, [eid_r, comp[1]])
                rz = plsc.load_gather(rows_v, [eid_r, comp[2]])
                cx = plsc.load_gather(rows_v, [eid_c, comp[0]])
                cy = plsc.load_gather(rows_v, [eid_c, comp[1]])
                cz = plsc.load_gather(rows_v, [eid_c, comp[2]])
                osl = pl.ds(e0b + t * _LANES, _LANES)
                dx = cx - rx
                dy = cy - ry
                dz = cz - rz
                s = dx * dx + dy * dy + dz * dz
                s_bits = lax.bitcast_convert_type(s, jnp.int32)
                y = lax.bitcast_convert_type(magic - (s_bits >> 1),
                                             jnp.float32)
                xh = s * half
                y = y * (threehalf - xh * y * y)
                y = y * (threehalf - xh * y * y)
                h = y * half
                ox_v[osl] = dx * h + half
                oy_v[osl] = dy * h + half
                oz_v[osl] = dz * h + half
            return _

        lax.fori_loop(0, chunk // _BLK, blk_body, None)
        for src, dst in out_parts(j):
            pltpu.make_async_copy(src, dst, sem_out.at[j % 2]).start()
        return _

    lax.fori_loop(0, n_mine, chunk_body, None)

    def drain_body(j, _):
        @pl.when(j >= lax.max(n_mine - 2, 0))
        def _():
            for src, dst in out_parts(j):
                pltpu.make_async_copy(src, dst, sem_out.at[j % 2]).wait()
        return _

    lax.fori_loop(lax.max(n_mine - 2, 0), n_mine, drain_body, None)


@functools.cache
def _build(n_edges: int):
    chunk = _pick_chunk(n_edges)
    n_chunks = n_edges // chunk
    mesh = plsc.VectorSubcoreMesh(core_axis_name="c", subcore_axis_name="s",
                                  num_cores=_NUM_CORES,
                                  num_subcores=_NUM_SUBCORES)
    comp = jax.ShapeDtypeStruct((n_edges,), jnp.float32)
    return pl.kernel(
        functools.partial(_sc_body, chunk=chunk, n_chunks=n_chunks),
        out_type=(comp, comp, comp),
        mesh=mesh,
        scratch_types=[
            pltpu.VMEM((chunk * 4,), jnp.int32),
            pltpu.VMEM((chunk * 4, 8), jnp.float32),
            pltpu.VMEM((chunk * 2,), jnp.float32),
            pltpu.VMEM((chunk * 2,), jnp.float32),
            pltpu.VMEM((chunk * 2,), jnp.float32),
            pltpu.SemaphoreType.DMA((2,)),
            pltpu.SemaphoreType.DMA((2,)),
            pltpu.SemaphoreType.DMA((2,)),
        ],
        compiler_params=pltpu.CompilerParams(needs_layout_passes=False,
                                             use_tc_tiling_on_sc=False),
    )


def kernel(pos, edge_index, edge_weight):
    n_edges = edge_weight.shape[0]
    pos4 = jnp.concatenate(
        [pos.astype(jnp.float32),
         jnp.zeros((pos.shape[0], 5), jnp.float32)], axis=1)
    ei_blk = (edge_index.astype(jnp.int32)
              .reshape(2, n_edges // _BLK, _BLK)
              .transpose(1, 0, 2)
              .reshape(2 * n_edges))
    ox, oy, oz = _build(n_edges)(pos4, ei_blk)
    return jnp.stack([ox, oy, oz, edge_weight.astype(jnp.float32)], axis=1)
